# Initial kernel scaffold; baseline (speedup 1.0000x reference)
import jax
import jax.numpy as jnp
from jax.experimental import pallas as pl


def kernel(nf, edge_index, initial_ef, W_edge, b_edge, bias):
    raise NotImplementedError("write your pallas kernel here")


# speedup vs baseline: 2.1035x; 2.1035x over previous
import functools

import jax
import jax.numpy as jnp
from jax import lax
from jax.experimental import pallas as pl
from jax.experimental.pallas import tpu as pltpu
from jax.experimental.pallas import tpu_sc as plsc

N_NODES = 10000
N_EDGES = 160000
HID = 32
D_EDGE = 16

NC = 2
NS = 16
NW = NC * NS
EPW = N_EDGES // NW
CH = 125
NCHUNK = EPW // CH
GRP = 8
GRP_ROWS = GRP * CH
NGRP = NCHUNK // GRP
RPT = 624
TAIL_BASE = NS * RPT
TAIL = N_NODES - TAIL_BASE

def _sc_mesh():
    return plsc.VectorSubcoreMesh(
        core_axis_name="c", subcore_axis_name="s", num_cores=NC, num_subcores=NS)


def _gather_body(nf_hbm, src_hbm, out_hbm, idx_v, rows_v, sem):
    c = lax.axis_index("c")
    s = lax.axis_index("s")
    wid = s * NC + c
    pltpu.sync_copy(src_hbm.at[pl.ds(wid * NCHUNK, NCHUNK)], idx_v)

    def group(g, carry):
        cps = [pltpu.async_copy(nf_hbm.at[idx_v.at[g * GRP + k]],
                                rows_v.at[pl.ds(k * CH, CH)], sem)
               for k in range(GRP)]
        for cp in cps:
            cp.wait()
        pltpu.sync_copy(rows_v, out_hbm.at[pl.ds(wid * EPW + g * GRP_ROWS,
                                                 GRP_ROWS)])
        return carry

    lax.fori_loop(0, NGRP, group, 0)


@jax.jit
def _gather(nf, src2d):
    return pl.kernel(
        _gather_body,
        out_type=jax.ShapeDtypeStruct((N_EDGES, HID), jnp.float32),
        mesh=_sc_mesh(),
        compiler_params=pltpu.CompilerParams(use_tc_tiling_on_sc=False),
        scratch_types=[
            pltpu.VMEM((NCHUNK, CH), jnp.int32),
            pltpu.VMEM((GRP_ROWS, HID), jnp.float32),
            pltpu.SemaphoreType.DMA,
        ],
    )(nf, src2d)


_MM_BLK = 2000


def _mm_body(x_ref, ef_ref, w_ref, m_ref):
    x = x_ref[...]
    ef = ef_ref[...]
    z = jnp.concatenate(
        [ef[:, d:d + 1] * x for d in range(D_EDGE)] + [x], axis=1)
    m_ref[...] = jnp.dot(z, w_ref[...], preferred_element_type=jnp.float32)


@jax.jit
def _matmul(x_src, ef, w_full):
    return pl.pallas_call(
        _mm_body,
        grid=(N_EDGES // _MM_BLK,),
        in_specs=[
            pl.BlockSpec((_MM_BLK, HID), lambda i: (i, 0)),
            pl.BlockSpec((_MM_BLK, D_EDGE), lambda i: (i, 0)),
            pl.BlockSpec(((D_EDGE + 1) * HID, HID), lambda i: (0, 0)),
        ],
        out_specs=pl.BlockSpec((_MM_BLK, HID), lambda i: (i, 0)),
        out_shape=jax.ShapeDtypeStruct((N_EDGES, HID), jnp.float32),
    )(x_src, ef, w_full)


def _scatter_body(m_hbm, dst_hbm, zeros_hbm, part_hbm, idx_v, rows_v, agg_sh, sem):
    c = lax.axis_index("c")
    s = lax.axis_index("s")
    wid = s * NC + c
    pltpu.sync_copy(zeros_hbm.at[pl.ds(s * RPT, RPT)],
                    agg_sh.at[pl.ds(s * RPT, RPT)])

    @pl.when(s == 0)
    def _():
        pltpu.sync_copy(zeros_hbm.at[pl.ds(TAIL_BASE, TAIL)],
                        agg_sh.at[pl.ds(TAIL_BASE, TAIL)])

    pltpu.sync_copy(dst_hbm.at[pl.ds(wid * NCHUNK, NCHUNK)], idx_v)
    plsc.subcore_barrier()

    def group(g, carry):
        pltpu.sync_copy(m_hbm.at[pl.ds(wid * EPW + g * GRP_ROWS, GRP_ROWS)],
                        rows_v)
        for k in range(GRP):
            pltpu.sync_copy(rows_v.at[pl.ds(k * CH, CH)],
                            agg_sh.at[idx_v.at[g * GRP + k]], add=True)
        return carry

    lax.fori_loop(0, NGRP, group, 0)
    plsc.subcore_barrier()
    pltpu.sync_copy(agg_sh.at[pl.ds(s * RPT, RPT)],
                    part_hbm.at[c].at[pl.ds(s * RPT, RPT)])

    @pl.when(s == 0)
    def _():
        pltpu.sync_copy(agg_sh.at[pl.ds(TAIL_BASE, TAIL)],
                        part_hbm.at[c].at[pl.ds(TAIL_BASE, TAIL)])


@jax.jit
def _scatter(m, dst2d, zeros):
    return pl.kernel(
        _scatter_body,
        out_type=jax.ShapeDtypeStruct((NC, N_NODES, HID), jnp.float32),
        mesh=_sc_mesh(),
        compiler_params=pltpu.CompilerParams(use_tc_tiling_on_sc=False),
        scratch_types=[
            pltpu.VMEM((NCHUNK, CH), jnp.int32),
            pltpu.VMEM((GRP_ROWS, HID), jnp.float32),
            pltpu.VMEM_SHARED((N_NODES, HID), jnp.float32),
            pltpu.SemaphoreType.DMA,
        ],
    )(m, dst2d, zeros)


def _comb_body(p_ref, nf_ref, b_ref, o_ref):
    o_ref[...] = p_ref[0] + p_ref[1] + nf_ref[...] + b_ref[...]


@jax.jit
def _combine(part, nf, bias2d):
    return pl.pallas_call(
        _comb_body,
        out_shape=jax.ShapeDtypeStruct((N_NODES, HID), jnp.float32),
    )(part, nf, bias2d)


def kernel(nf, edge_index, initial_ef, W_edge, b_edge, bias):
    src2d = edge_index[0].astype(jnp.int32).reshape(N_EDGES // CH, CH)
    dst2d = edge_index[1].astype(jnp.int32).reshape(N_EDGES // CH, CH)
    w_full = jnp.concatenate(
        [W_edge.reshape(D_EDGE * HID, HID), b_edge.reshape(HID, HID)], axis=0)
    zeros = jnp.zeros((N_NODES, HID), jnp.float32)

    x_src = _gather(nf, src2d)
    m = _matmul(x_src, initial_ef, w_full)
    part = _scatter(m, dst2d, zeros)
    return _combine(part, nf, bias.reshape(1, HID))


# speedup vs baseline: 2.8231x; 1.3421x over previous
import functools

import jax
import jax.numpy as jnp
from jax import lax
from jax.experimental import pallas as pl
from jax.experimental.pallas import tpu as pltpu
from jax.experimental.pallas import tpu_sc as plsc

N_NODES = 10000
N_EDGES = 160000
HID = 32
D_EDGE = 16

NC = 2
NS = 16
NW = NC * NS
EPW = N_EDGES // NW
CH = 125
NCHUNK = EPW // CH
GRP = 8
GRP_ROWS = GRP * CH
NGRP = NCHUNK // GRP
RPT = 624
TAIL_BASE = NS * RPT
TAIL = N_NODES - TAIL_BASE

def _sc_mesh():
    return plsc.VectorSubcoreMesh(
        core_axis_name="c", subcore_axis_name="s", num_cores=NC, num_subcores=NS)


def _gather_body(nf_hbm, src_hbm, out_hbm, idx_v, rows_v, sem):
    c = lax.axis_index("c")
    s = lax.axis_index("s")
    wid = s * NC + c
    pltpu.sync_copy(src_hbm.at[pl.ds(wid * NCHUNK, NCHUNK)], idx_v)

    def group(g, carry):
        cps = [pltpu.async_copy(nf_hbm.at[idx_v.at[g * GRP + k]],
                                rows_v.at[pl.ds(k * CH, CH)], sem)
               for k in range(GRP)]
        for cp in cps:
            cp.wait()
        pltpu.sync_copy(rows_v, out_hbm.at[pl.ds(wid * EPW + g * GRP_ROWS,
                                                 GRP_ROWS)])
        return carry

    lax.fori_loop(0, NGRP, group, 0)


@jax.jit
def _gather(nf, src2d):
    return pl.kernel(
        _gather_body,
        out_type=jax.ShapeDtypeStruct((N_EDGES, HID), jnp.float32),
        mesh=_sc_mesh(),
        compiler_params=pltpu.CompilerParams(use_tc_tiling_on_sc=False),
        scratch_types=[
            pltpu.VMEM((NCHUNK, CH), jnp.int32),
            pltpu.VMEM((GRP_ROWS, HID), jnp.float32),
            pltpu.SemaphoreType.DMA,
        ],
    )(nf, src2d)


_MM_BLK = 2000


def _mm_body(x_ref, ef_ref, w_ref, m_ref):
    x = x_ref[...]
    ef = ef_ref[...]
    acc = jnp.dot(x, w_ref[D_EDGE], preferred_element_type=jnp.float32)
    for d in range(D_EDGE):
        acc += ef[:, d:d + 1] * jnp.dot(x, w_ref[d],
                                        preferred_element_type=jnp.float32)
    m_ref[...] = acc


@jax.jit
def _matmul(x_src, ef, w_full):
    return pl.pallas_call(
        _mm_body,
        grid=(N_EDGES // _MM_BLK,),
        in_specs=[
            pl.BlockSpec((_MM_BLK, HID), lambda i: (i, 0)),
            pl.BlockSpec((_MM_BLK, D_EDGE), lambda i: (i, 0)),
            pl.BlockSpec((D_EDGE + 1, HID, HID), lambda i: (0, 0, 0)),
        ],
        out_specs=pl.BlockSpec((_MM_BLK, HID), lambda i: (i, 0)),
        out_shape=jax.ShapeDtypeStruct((N_EDGES, HID), jnp.float32),
    )(x_src, ef, w_full)


def _scatter_body(m_hbm, dst_hbm, zeros_hbm, part_hbm, idx_v, rows_v, agg_sh, sem):
    c = lax.axis_index("c")
    s = lax.axis_index("s")
    wid = s * NC + c
    pltpu.sync_copy(zeros_hbm.at[pl.ds(s * RPT, RPT)],
                    agg_sh.at[pl.ds(s * RPT, RPT)])

    @pl.when(s == 0)
    def _():
        pltpu.sync_copy(zeros_hbm.at[pl.ds(TAIL_BASE, TAIL)],
                        agg_sh.at[pl.ds(TAIL_BASE, TAIL)])

    pltpu.sync_copy(dst_hbm.at[pl.ds(wid * NCHUNK, NCHUNK)], idx_v)
    plsc.subcore_barrier()

    def group(g, carry):
        pltpu.sync_copy(m_hbm.at[pl.ds(wid * EPW + g * GRP_ROWS, GRP_ROWS)],
                        rows_v)
        for k in range(GRP):
            pltpu.sync_copy(rows_v.at[pl.ds(k * CH, CH)],
                            agg_sh.at[idx_v.at[g * GRP + k]], add=True)
        return carry

    lax.fori_loop(0, NGRP, group, 0)
    plsc.subcore_barrier()
    pltpu.sync_copy(agg_sh.at[pl.ds(s * RPT, RPT)],
                    part_hbm.at[c].at[pl.ds(s * RPT, RPT)])

    @pl.when(s == 0)
    def _():
        pltpu.sync_copy(agg_sh.at[pl.ds(TAIL_BASE, TAIL)],
                        part_hbm.at[c].at[pl.ds(TAIL_BASE, TAIL)])


@jax.jit
def _scatter(m, dst2d, zeros):
    return pl.kernel(
        _scatter_body,
        out_type=jax.ShapeDtypeStruct((NC, N_NODES, HID), jnp.float32),
        mesh=_sc_mesh(),
        compiler_params=pltpu.CompilerParams(use_tc_tiling_on_sc=False),
        scratch_types=[
            pltpu.VMEM((NCHUNK, CH), jnp.int32),
            pltpu.VMEM((GRP_ROWS, HID), jnp.float32),
            pltpu.VMEM_SHARED((N_NODES, HID), jnp.float32),
            pltpu.SemaphoreType.DMA,
        ],
    )(m, dst2d, zeros)


def _comb_body(p_ref, nf_ref, b_ref, o_ref):
    o_ref[...] = p_ref[0] + p_ref[1] + nf_ref[...] + b_ref[...]


@jax.jit
def _combine(part, nf, bias2d):
    return pl.pallas_call(
        _comb_body,
        out_shape=jax.ShapeDtypeStruct((N_NODES, HID), jnp.float32),
    )(part, nf, bias2d)


def kernel(nf, edge_index, initial_ef, W_edge, b_edge, bias):
    src2d = edge_index[0].astype(jnp.int32).reshape(N_EDGES // CH, CH)
    dst2d = edge_index[1].astype(jnp.int32).reshape(N_EDGES // CH, CH)
    w_full = jnp.concatenate(
        [W_edge.reshape(D_EDGE * HID, HID), b_edge.reshape(HID, HID)],
        axis=0).reshape(D_EDGE + 1, HID, HID)
    zeros = jnp.zeros((N_NODES, HID), jnp.float32)

    x_src = _gather(nf, src2d)
    m = _matmul(x_src, initial_ef, w_full)
    part = _scatter(m, dst2d, zeros)
    return _combine(part, nf, bias.reshape(1, HID))
